# baseline (speedup 1.0000x reference)
import functools

import jax
import jax.numpy as jnp
from jax import lax
from jax.experimental import pallas as pl
from jax.experimental.pallas import tpu as pltpu
from jax.experimental.pallas import tpu_sc as plsc


@functools.cache
def _make_broadcast_kernel(batch, seq_len, dim, dtype):
    info = plsc.get_sparse_core_info()
    num_workers = info.num_cores * info.num_subcores
    num_cores = info.num_cores
    assert seq_len % num_workers == 0
    rows_per_worker = seq_len // num_workers
    chunk = min(64, rows_per_worker)
    assert rows_per_worker % chunk == 0
    n_chunks = rows_per_worker // chunk

    mesh = plsc.VectorSubcoreMesh(core_axis_name="c", subcore_axis_name="s")

    @functools.partial(
        pl.kernel,
        out_type=jax.ShapeDtypeStruct((batch, seq_len, dim), dtype),
        mesh=mesh,
        scratch_types=[pltpu.VMEM((chunk, dim), dtype)],
    )
    def bcast(w_hbm, out_hbm, buf):
        wid = lax.axis_index("s") * num_cores + lax.axis_index("c")
        base = wid * rows_per_worker
        for c in range(n_chunks):
            r0 = base + c * chunk
            pltpu.sync_copy(w_hbm.at[pl.ds(r0, chunk)], buf)
            for b in range(batch):
                pltpu.sync_copy(buf, out_hbm.at[b, pl.ds(r0, chunk)])

    return bcast


def kernel(x, weight):
    batch, seq_len, dim = x.shape
    return _make_broadcast_kernel(batch, seq_len, dim, weight.dtype)(weight)
